# baseline (speedup 1.0000x reference)
import jax
import jax.numpy as jnp
from jax import lax
from jax.experimental import pallas as pl
from jax.experimental.pallas import tpu as pltpu

_PARALLEL = pltpu.CompilerParams(dimension_semantics=("parallel",))

N, D, H, Z2 = 4096, 128, 128, 64
BM = 512
NEG = -1000000000000.0


def _leaky(v):
    return jnp.maximum(v, 0.25 * v)


def _bdot(x, y):
    return jnp.dot(x.astype(jnp.bfloat16), y.astype(jnp.bfloat16),
                   preferred_element_type=jnp.float32)


def _mm_kernel(x_ref, w_ref, o_ref):
    o_ref[:] = _bdot(x_ref[:], w_ref[:])


def _mm(x, w):
    m, k = x.shape
    _, n = w.shape
    bm = 1024
    return pl.pallas_call(
        _mm_kernel,
        grid=(m // bm,),
        in_specs=[
            pl.BlockSpec((bm, k), lambda i: (i, 0)),
            pl.BlockSpec((k, n), lambda i: (0, 0)),
        ],
        out_specs=pl.BlockSpec((bm, n), lambda i: (i, 0)),
        out_shape=jax.ShapeDtypeStruct((m, n), jnp.float32),
        compiler_params=_PARALLEL,
    )(x, w)


def _gcn1_kernel(adj_ref, s_ref, b_ref, w_ref, o_ref):
    acc = _bdot(adj_ref[:], s_ref[:])
    t = _leaky(acc + b_ref[:])
    o_ref[:] = _bdot(t, w_ref[:])


def _gcn1(adj, support, b, w_next):
    h = support.shape[1]
    hn = w_next.shape[1]
    return pl.pallas_call(
        _gcn1_kernel,
        grid=(N // BM,),
        in_specs=[
            pl.BlockSpec((BM, N), lambda i: (i, 0)),
            pl.BlockSpec((N, h), lambda i: (0, 0)),
            pl.BlockSpec((1, h), lambda i: (0, 0)),
            pl.BlockSpec((h, hn), lambda i: (0, 0)),
        ],
        out_specs=pl.BlockSpec((BM, hn), lambda i: (i, 0)),
        out_shape=jax.ShapeDtypeStruct((N, hn), jnp.float32),
        compiler_params=_PARALLEL,
    )(adj, support, b, w_next)


def _gcn2_kernel(adj_ref, s_ref, b_ref, w_ref, a1_ref, a2_ref,
                 h_ref, s1_ref, s2t_ref, m_ref):
    adj = adj_ref[:]
    acc = _bdot(adj, s_ref[:])
    t = _leaky(acc + b_ref[:])
    hb = _bdot(t, w_ref[:])
    h_ref[:] = hb
    s1_ref[:] = _bdot(hb, a1_ref[:])
    s2t_ref[:] = lax.dot_general(
        a2_ref[:].astype(jnp.bfloat16), hb.astype(jnp.bfloat16),
        (((1,), (1,)), ((), ())),
        preferred_element_type=jnp.float32)
    m_ref[:] = jnp.where(adj > 0, jnp.float32(1.0), jnp.float32(0.0)
                         ).astype(jnp.int8)


def _gcn2(adj, support, b, w_next, a1c, a2r):
    h = support.shape[1]
    hn = w_next.shape[1]
    return pl.pallas_call(
        _gcn2_kernel,
        grid=(N // BM,),
        in_specs=[
            pl.BlockSpec((BM, N), lambda i: (i, 0)),
            pl.BlockSpec((N, h), lambda i: (0, 0)),
            pl.BlockSpec((1, h), lambda i: (0, 0)),
            pl.BlockSpec((h, hn), lambda i: (0, 0)),
            pl.BlockSpec((hn, 1), lambda i: (0, 0)),
            pl.BlockSpec((1, hn), lambda i: (0, 0)),
        ],
        out_specs=[
            pl.BlockSpec((BM, hn), lambda i: (i, 0)),
            pl.BlockSpec((BM, 1), lambda i: (i, 0)),
            pl.BlockSpec((1, BM), lambda i: (0, i)),
            pl.BlockSpec((BM, N), lambda i: (i, 0)),
        ],
        out_shape=[
            jax.ShapeDtypeStruct((N, hn), jnp.float32),
            jax.ShapeDtypeStruct((N, 1), jnp.float32),
            jax.ShapeDtypeStruct((1, N), jnp.float32),
            jax.ShapeDtypeStruct((N, N), jnp.int8),
        ],
        compiler_params=_PARALLEL,
    )(adj, support, b, w_next, a1c, a2r)


def _gat_kernel(m_ref, h_ref, s1_ref, s2t_ref, o_ref):
    e = _leaky(s1_ref[:] + s2t_ref[:])
    e = jnp.where(m_ref[:].astype(jnp.float32) > 0, e, NEG)
    m = jnp.max(e, axis=1, keepdims=True)
    p = jnp.exp(e - m)
    l = jnp.sum(p, axis=1, keepdims=True)
    o = _bdot(p / l, h_ref[:])
    o_ref[:] = _leaky(o)


def _gat(mask, h, s1, s2t):
    return pl.pallas_call(
        _gat_kernel,
        grid=(N // BM,),
        in_specs=[
            pl.BlockSpec((BM, N), lambda i: (i, 0)),
            pl.BlockSpec((N, Z2), lambda i: (0, 0)),
            pl.BlockSpec((BM, 1), lambda i: (i, 0)),
            pl.BlockSpec((1, N), lambda i: (0, 0)),
        ],
        out_specs=pl.BlockSpec((BM, Z2), lambda i: (i, 0)),
        out_shape=jax.ShapeDtypeStruct((N, Z2), jnp.float32),
        compiler_params=_PARALLEL,
    )(mask, h, s1, s2t)


def kernel(x, adj, W1, b1, W2, b2, Wg, a):
    b1r = b1.reshape(1, H)
    b2r = b2.reshape(1, H)
    a1c = a[:Z2].reshape(Z2, 1)
    a2r = a[Z2:, 0].reshape(1, Z2)
    support1 = _mm(x, W1)
    support2 = _gcn1(adj, support1, b1r, W2)
    h, s1, s2t, mask = _gcn2(adj, support2, b2r, Wg, a1c, a2r)
    out = _gat(mask, h, s1, s2t)
    return out[:, : Z2 // 2], out[:, Z2 // 2 :]


# speedup vs baseline: 1.0120x; 1.0120x over previous
import jax
import jax.numpy as jnp
from jax import lax
from jax.experimental import pallas as pl
from jax.experimental.pallas import tpu as pltpu

_PARALLEL = pltpu.CompilerParams(dimension_semantics=("parallel",))

N, D, H, Z2 = 4096, 128, 128, 64
BM = 1024
NEG = -1000000000000.0


def _leaky(v):
    return jnp.maximum(v, 0.25 * v)


def _bdot(x, y):
    return jnp.dot(x.astype(jnp.bfloat16), y.astype(jnp.bfloat16),
                   preferred_element_type=jnp.float32)


def _mm_kernel(x_ref, w_ref, o_ref):
    o_ref[:] = _bdot(x_ref[:], w_ref[:])


def _mm(x, w):
    m, k = x.shape
    _, n = w.shape
    bm = 1024
    return pl.pallas_call(
        _mm_kernel,
        grid=(m // bm,),
        in_specs=[
            pl.BlockSpec((bm, k), lambda i: (i, 0)),
            pl.BlockSpec((k, n), lambda i: (0, 0)),
        ],
        out_specs=pl.BlockSpec((bm, n), lambda i: (i, 0)),
        out_shape=jax.ShapeDtypeStruct((m, n), jnp.float32),
        compiler_params=_PARALLEL,
    )(x, w)


def _gcn1_kernel(adj_ref, s_ref, b_ref, w_ref, o_ref):
    acc = _bdot(adj_ref[:], s_ref[:])
    t = _leaky(acc + b_ref[:])
    o_ref[:] = _bdot(t, w_ref[:])


def _gcn1(adj, support, b, w_next):
    h = support.shape[1]
    hn = w_next.shape[1]
    return pl.pallas_call(
        _gcn1_kernel,
        grid=(N // BM,),
        in_specs=[
            pl.BlockSpec((BM, N), lambda i: (i, 0)),
            pl.BlockSpec((N, h), lambda i: (0, 0)),
            pl.BlockSpec((1, h), lambda i: (0, 0)),
            pl.BlockSpec((h, hn), lambda i: (0, 0)),
        ],
        out_specs=pl.BlockSpec((BM, hn), lambda i: (i, 0)),
        out_shape=jax.ShapeDtypeStruct((N, hn), jnp.float32),
        compiler_params=_PARALLEL,
    )(adj, support, b, w_next)


def _gcn2_kernel(adj_ref, s_ref, b_ref, w_ref, a1_ref, a2_ref,
                 h_ref, s1_ref, s2t_ref, m_ref):
    adj = adj_ref[:]
    acc = _bdot(adj, s_ref[:])
    t = _leaky(acc + b_ref[:])
    hb = _bdot(t, w_ref[:])
    h_ref[:] = hb
    s1_ref[:] = _bdot(hb, a1_ref[:])
    s2t_ref[:] = lax.dot_general(
        a2_ref[:].astype(jnp.bfloat16), hb.astype(jnp.bfloat16),
        (((1,), (1,)), ((), ())),
        preferred_element_type=jnp.float32)
    m_ref[:] = jnp.where(adj > 0, jnp.float32(1.0), jnp.float32(0.0)
                         ).astype(jnp.int8)


def _gcn2(adj, support, b, w_next, a1c, a2r):
    h = support.shape[1]
    hn = w_next.shape[1]
    return pl.pallas_call(
        _gcn2_kernel,
        grid=(N // BM,),
        in_specs=[
            pl.BlockSpec((BM, N), lambda i: (i, 0)),
            pl.BlockSpec((N, h), lambda i: (0, 0)),
            pl.BlockSpec((1, h), lambda i: (0, 0)),
            pl.BlockSpec((h, hn), lambda i: (0, 0)),
            pl.BlockSpec((hn, 1), lambda i: (0, 0)),
            pl.BlockSpec((1, hn), lambda i: (0, 0)),
        ],
        out_specs=[
            pl.BlockSpec((BM, hn), lambda i: (i, 0)),
            pl.BlockSpec((BM, 1), lambda i: (i, 0)),
            pl.BlockSpec((1, BM), lambda i: (0, i)),
            pl.BlockSpec((BM, N), lambda i: (i, 0)),
        ],
        out_shape=[
            jax.ShapeDtypeStruct((N, hn), jnp.float32),
            jax.ShapeDtypeStruct((N, 1), jnp.float32),
            jax.ShapeDtypeStruct((1, N), jnp.float32),
            jax.ShapeDtypeStruct((N, N), jnp.int8),
        ],
        compiler_params=_PARALLEL,
    )(adj, support, b, w_next, a1c, a2r)


def _gat_kernel(m_ref, h_ref, s1_ref, s2t_ref, o_ref):
    e = _leaky(s1_ref[:] + s2t_ref[:])
    e = jnp.where(m_ref[:].astype(jnp.float32) > 0, e, NEG)
    m = jnp.max(e, axis=1, keepdims=True)
    p = jnp.exp(e - m)
    l = jnp.sum(p, axis=1, keepdims=True)
    o = _bdot(p / l, h_ref[:])
    o_ref[:] = _leaky(o)


def _gat(mask, h, s1, s2t):
    return pl.pallas_call(
        _gat_kernel,
        grid=(N // BM,),
        in_specs=[
            pl.BlockSpec((BM, N), lambda i: (i, 0)),
            pl.BlockSpec((N, Z2), lambda i: (0, 0)),
            pl.BlockSpec((BM, 1), lambda i: (i, 0)),
            pl.BlockSpec((1, N), lambda i: (0, 0)),
        ],
        out_specs=pl.BlockSpec((BM, Z2), lambda i: (i, 0)),
        out_shape=jax.ShapeDtypeStruct((N, Z2), jnp.float32),
        compiler_params=_PARALLEL,
    )(mask, h, s1, s2t)


def kernel(x, adj, W1, b1, W2, b2, Wg, a):
    b1r = b1.reshape(1, H)
    b2r = b2.reshape(1, H)
    a1c = a[:Z2].reshape(Z2, 1)
    a2r = a[Z2:, 0].reshape(1, Z2)
    support1 = _mm(x, W1)
    support2 = _gcn1(adj, support1, b1r, W2)
    h, s1, s2t, mask = _gcn2(adj, support2, b2r, Wg, a1c, a2r)
    out = _gat(mask, h, s1, s2t)
    return out[:, : Z2 // 2], out[:, Z2 // 2 :]
